# Initial kernel scaffold; baseline (speedup 1.0000x reference)
import jax
import jax.numpy as jnp
from jax.experimental import pallas as pl


def kernel(x, table, W1, b1, W2, b2):
    raise NotImplementedError("write your pallas kernel here")


# speedup vs baseline: 2.5479x; 2.5479x over previous
import functools

import jax
import jax.numpy as jnp
from jax import lax
from jax.experimental import pallas as pl
from jax.experimental.pallas import tpu as pltpu
from jax.experimental.pallas import tpu_sc as plsc

_NC = 2
_NS = 16
_NW = _NC * _NS
_LANES = 16


@functools.lru_cache(maxsize=None)
def _make_sc_gather_sum(B, L, V, D):
    L2 = 2 * L
    PAIRS = B // 2 // _NW
    ROWS = B // _NW
    NBUF = 4
    NVR = D // _LANES
    UNROLL = 5
    assert B % (2 * _NW) == 0 and PAIRS % NBUF == 0
    assert L2 <= 128 and D % _LANES == 0 and L % UNROLL == 0

    mesh = plsc.VectorSubcoreMesh(core_axis_name="c", subcore_axis_name="s")

    @functools.partial(
        pl.kernel,
        mesh=mesh,
        compiler_params=pltpu.CompilerParams(use_tc_tiling_on_sc=False),
        out_type=jax.ShapeDtypeStruct((B, D), jnp.float32),
        scratch_types=[
            pltpu.VMEM((PAIRS, L2), jnp.int32),
            pltpu.VMEM((NBUF, L2, D), jnp.float32),
            pltpu.VMEM((ROWS, D), jnp.float32),
        ] + [pltpu.SemaphoreType.DMA] * NBUF,
    )
    def sc_gather_sum(x2_hbm, table_hbm, out_hbm, idx_v, rows_v, acc_v, *sems):
        wid = lax.axis_index("s") * _NC + lax.axis_index("c")
        pbase = wid * PAIRS
        pltpu.sync_copy(x2_hbm.at[pl.ds(pbase, PAIRS)], idx_v)

        def start(j, b):
            pltpu.async_copy(table_hbm.at[idx_v.at[j]], rows_v.at[b], sems[b])

        for b in range(NBUF):
            start(b, b)

        def outer(g, carry):
            for b in range(NBUF):
                j = g * NBUF + b
                pltpu.make_async_copy(
                    table_hbm.at[idx_v.at[j]], rows_v.at[b], sems[b]).wait()
                jn = j + NBUF

                @pl.when(jn < PAIRS)
                def _():
                    start(jn, b)

                for half in range(2):
                    def body(i, accs, half=half, b=b):
                        out = list(accs)
                        for u in range(UNROLL):
                            r = half * L + i * UNROLL + u
                            for q in range(NVR):
                                out[q] = out[q] + rows_v[
                                    b, r, pl.ds(q * _LANES, _LANES)]
                        return tuple(out)

                    zero = jnp.zeros((_LANES,), jnp.float32)
                    accs = lax.fori_loop(0, L // UNROLL, body, (zero,) * NVR)
                    row = 2 * j + half
                    for q in range(NVR):
                        acc_v[row, pl.ds(q * _LANES, _LANES)] = accs[q]
            return carry

        lax.fori_loop(0, PAIRS // NBUF, outer, 0)
        pltpu.sync_copy(acc_v, out_hbm.at[pl.ds(wid * ROWS, ROWS)])

    return sc_gather_sum


@functools.lru_cache(maxsize=None)
def _make_tc_mlp(B, L, D, H, C):
    BLK = 512
    assert B % BLK == 0

    def body(x_ref, s_ref, w1_ref, b1_ref, w2_ref, b2_ref, o_ref):
        cnt = jnp.sum((x_ref[...] != 0).astype(jnp.float32), axis=1,
                      keepdims=True)
        avg = s_ref[...] / jnp.maximum(cnt, 1e-6)
        h = jnp.dot(avg, w1_ref[...], preferred_element_type=jnp.float32)
        h = jnp.maximum(h + b1_ref[...], 0.0)
        o_ref[...] = (jnp.dot(h, w2_ref[...],
                              preferred_element_type=jnp.float32)
                      + b2_ref[...])

    return pl.pallas_call(
        body,
        grid=(B // BLK,),
        in_specs=[
            pl.BlockSpec((BLK, L), lambda i: (i, 0)),
            pl.BlockSpec((BLK, D), lambda i: (i, 0)),
            pl.BlockSpec((D, H), lambda i: (0, 0)),
            pl.BlockSpec((1, H), lambda i: (0, 0)),
            pl.BlockSpec((H, C), lambda i: (0, 0)),
            pl.BlockSpec((1, C), lambda i: (0, 0)),
        ],
        out_specs=pl.BlockSpec((BLK, C), lambda i: (i, 0)),
        out_shape=jax.ShapeDtypeStruct((B, C), jnp.float32),
    )


def kernel(x, table, W1, b1, W2, b2):
    B, L = x.shape
    V, D = table.shape
    H = W1.shape[1]
    C = W2.shape[1]
    xi = x.astype(jnp.int32)
    x2 = xi.reshape(B // 2, 2 * L)
    summed = _make_sc_gather_sum(B, L, V, D)(x2, table)
    out = _make_tc_mlp(B, L, D, H, C)(
        xi, summed, W1, b1.reshape(1, H), W2, b2.reshape(1, C))
    return out
